# baseline (speedup 1.0000x reference)
import numpy as np
import jax
import jax.numpy as jnp
from jax.experimental import pallas as pl
from jax.experimental.pallas import tpu as pltpu

N = 1024
R = 4
IN_DIM = 256
HID = 256
D = IN_DIM * R
BF = jnp.bfloat16
F32 = jnp.float32

_a = np.arange(D)
_S3_np = np.zeros((R, D, IN_DIM), dtype=np.float32)
for _r in range(R):
    _S3_np[_r, _a[_a % R == _r], (_a[_a % R == _r] // R)] = 1.0


def _body(adj_ref, xf_ref, hf_ref, s3_ref,
          wxz_ref, wxr_ref, wxh_ref, whz_ref, whr_ref, out_ref):
    xf16 = xf_ref[...].astype(BF)
    hf16 = hf_ref[...].astype(BF)
    acc = None
    for r in range(R):
        s3 = s3_ref[r]
        xd = jnp.dot(xf16, s3, preferred_element_type=F32).astype(BF)
        h32 = jnp.dot(hf16, s3, preferred_element_type=F32)
        hd = h32.astype(BF)
        a16 = adj_ref[r].astype(BF)
        ax = jnp.dot(a16, xd, preferred_element_type=F32).astype(BF)
        ah = jnp.dot(a16, hd, preferred_element_type=F32).astype(BF)
        zpre = (jnp.dot(ax, wxz_ref[r].astype(BF), preferred_element_type=F32)
                + jnp.dot(ah, whz_ref[r].astype(BF), preferred_element_type=F32))
        rpre = (jnp.dot(ax, wxr_ref[r].astype(BF), preferred_element_type=F32)
                + jnp.dot(ah, whr_ref[r].astype(BF), preferred_element_type=F32))
        z = jax.nn.sigmoid(zpre)
        rg = jax.nn.sigmoid(rpre)
        tterm = jnp.dot(ax, wxh_ref[r].astype(BF), preferred_element_type=F32)
        g16 = (rg * h32).astype(BF)
        ag = jnp.dot(a16, g16, preferred_element_type=F32)
        ht = jnp.tanh(tterm + jnp.dot(ag.astype(BF), whr_ref[r].astype(BF),
                                      preferred_element_type=F32))
        hn = (z * h32 + (1.0 - z) * ht).astype(BF)
        part = jax.lax.dot_general(hn, s3, (((1,), (1,)), ((), ())),
                                   preferred_element_type=F32)
        acc = part if acc is None else acc + part
    out_ref[...] = acc


def kernel(X, adj, h_pre, W_xz, W_xr, W_xh, W_hz, W_hr, W_hh):
    del W_hh
    Xf = X.reshape(N, D)
    Hf = h_pre.reshape(N, D)
    S3 = jnp.asarray(_S3_np, dtype=BF)
    full = lambda *shape: pl.BlockSpec(shape, lambda: tuple(0 for _ in shape))
    out = pl.pallas_call(
        _body,
        grid=(),
        in_specs=[
            full(R, N, N),
            full(N, D),
            full(N, D),
            full(R, D, IN_DIM),
            full(R, IN_DIM, HID),
            full(R, IN_DIM, HID),
            full(R, IN_DIM, HID),
            full(R, HID, HID),
            full(R, HID, HID),
        ],
        out_specs=full(N, HID * R),
        out_shape=jax.ShapeDtypeStruct((N, HID * R), F32),
    )(adj, Xf, Hf, S3, W_xz, W_xr, W_xh, W_hz, W_hr)
    return out.reshape(N, HID, R)
